# Initial kernel scaffold; baseline (speedup 1.0000x reference)
import jax
import jax.numpy as jnp
from jax.experimental import pallas as pl


def kernel(x, gate_w, gate_b, W1, b1, W2, b2, W3, b3, Ws1, bs1, Ws2, bs2, Ws3, bs3):
    raise NotImplementedError("write your pallas kernel here")


# speedup vs baseline: 1.4207x; 1.4207x over previous
import functools

import jax
import jax.numpy as jnp
from jax.experimental import pallas as pl
from jax.experimental.pallas import tpu as pltpu

E = 8
TOPK = 2
D = 1024
FF = 512
NS = 2
ROUTE_SCALE = 1.0

EA = E + NS
TB = 512


def _moe_body(x_ref, gw_ref, gb_ref, w1_ref, b1_ref, w3_ref, b3_ref,
              w2_ref, b2_ref, out_ref, comb_ref):
    e = pl.program_id(1)

    @pl.when(e == 0)
    def _compute_gate():
        xb = x_ref[...]
        scores = jax.lax.dot_general(
            xb, gw_ref[...], (((1,), (1,)), ((), ())),
            preferred_element_type=jnp.float32)
        scores = jax.nn.softmax(scores, axis=-1)
        biased = scores + gb_ref[...]
        lanes = jax.lax.broadcasted_iota(jnp.int32, (TB, E), 1)
        i1 = jnp.argmax(biased, axis=-1)[:, None]
        w1 = jnp.sum(jnp.where(lanes == i1, scores, 0.0), axis=-1, keepdims=True)
        masked = jnp.where(lanes == i1, -jnp.inf, biased)
        i2 = jnp.argmax(masked, axis=-1)[:, None]
        w2 = jnp.sum(jnp.where(lanes == i2, scores, 0.0), axis=-1, keepdims=True)
        comb = (jnp.where(lanes == i1, w1, 0.0) +
                jnp.where(lanes == i2, w2, 0.0)) * ROUTE_SCALE
        comb_ref[:, :E] = comb
        comb_ref[:, E:] = jnp.ones((TB, EA - E), jnp.float32)

    xb = x_ref[...]
    h1 = jax.lax.dot_general(
        xb, w1_ref[0], (((1,), (1,)), ((), ())),
        preferred_element_type=jnp.float32) + b1_ref[0]
    h3 = jax.lax.dot_general(
        xb, w3_ref[0], (((1,), (1,)), ((), ())),
        preferred_element_type=jnp.float32) + b3_ref[0]
    h = (h1 * jax.nn.sigmoid(h1)) * h3
    ye = jax.lax.dot_general(
        h, w2_ref[0], (((1,), (1,)), ((), ())),
        preferred_element_type=jnp.float32) + b2_ref[0]
    lane = jax.lax.broadcasted_iota(jnp.int32, (TB, EA), 1)
    col = jnp.sum(jnp.where(lane == e, comb_ref[...], 0.0),
                  axis=1, keepdims=True)
    contrib = ye * col

    @pl.when(e == 0)
    def _init():
        out_ref[...] = contrib

    @pl.when(e != 0)
    def _acc():
        out_ref[...] += contrib


@jax.jit
def _moe(x2, gate_w, gate_b2, W1a, b1a, W3a, b3a, W2a, b2a):
    T = x2.shape[0]
    grid = (T // TB, EA)
    return pl.pallas_call(
        _moe_body,
        grid=grid,
        in_specs=[
            pl.BlockSpec((TB, D), lambda t, e: (t, 0)),
            pl.BlockSpec((E, D), lambda t, e: (0, 0)),
            pl.BlockSpec((1, E), lambda t, e: (0, 0)),
            pl.BlockSpec((1, FF, D), lambda t, e: (e, 0, 0)),
            pl.BlockSpec((1, 1, FF), lambda t, e: (e, 0, 0)),
            pl.BlockSpec((1, FF, D), lambda t, e: (e, 0, 0)),
            pl.BlockSpec((1, 1, FF), lambda t, e: (e, 0, 0)),
            pl.BlockSpec((1, D, FF), lambda t, e: (e, 0, 0)),
            pl.BlockSpec((1, 1, D), lambda t, e: (e, 0, 0)),
        ],
        out_specs=pl.BlockSpec((TB, D), lambda t, e: (t, 0)),
        out_shape=jax.ShapeDtypeStruct((T, D), jnp.float32),
        scratch_shapes=[pltpu.VMEM((TB, EA), jnp.float32)],
        compiler_params=pltpu.CompilerParams(
            dimension_semantics=("parallel", "arbitrary"),
        ),
    )(x2, gate_w, gate_b2, W1a, b1a, W3a, b3a, W2a, b2a)


def kernel(x, gate_w, gate_b, W1, b1, W2, b2, W3, b3,
           Ws1, bs1, Ws2, bs2, Ws3, bs3):
    shape = x.shape
    x2 = x.reshape(-1, D)

    W1a = jnp.concatenate([W1, Ws1.reshape(NS, FF, D)], axis=0)
    b1a = jnp.concatenate([b1, bs1.reshape(NS, FF)], axis=0).reshape(EA, 1, FF)
    W3a = jnp.concatenate([W3, Ws3.reshape(NS, FF, D)], axis=0)
    b3a = jnp.concatenate([b3, bs3.reshape(NS, FF)], axis=0).reshape(EA, 1, FF)
    Ws2s = Ws2.reshape(D, NS, FF).transpose(1, 0, 2)
    W2a = jnp.concatenate([W2, Ws2s], axis=0)
    b2a = jnp.concatenate(
        [b2, bs2[None, :], jnp.zeros((NS - 1, D), jnp.float32)],
        axis=0).reshape(EA, 1, D)

    y = _moe(x2, gate_w, gate_b.reshape(1, E), W1a, b1a, W3a, b3a, W2a, b2a)
    return y.reshape(shape)


# speedup vs baseline: 1.6334x; 1.1498x over previous
import jax
import jax.numpy as jnp
from jax.experimental import pallas as pl
from jax.experimental.pallas import tpu as pltpu

E = 8
TOPK = 2
D = 1024
FF = 512
NS = 2
ROUTE_SCALE = 1.0

EA = E + NS
TB = 1024


def _moe_body(x_ref, xb16_ref, gw_ref, gb_ref, w1_ref, b1_ref, w3_ref, b3_ref,
              w2_ref, b2_ref, out_ref, comb_ref):
    e = pl.program_id(1)

    @pl.when(e == 0)
    def _compute_gate():
        xb = x_ref[...]
        scores = jax.lax.dot_general(
            xb, gw_ref[...], (((1,), (1,)), ((), ())),
            preferred_element_type=jnp.float32)
        scores = jax.nn.softmax(scores, axis=-1)
        biased = scores + gb_ref[...]
        lanes = jax.lax.broadcasted_iota(jnp.int32, (TB, E), 1)
        i1 = jnp.argmax(biased, axis=-1)[:, None]
        w1 = jnp.sum(jnp.where(lanes == i1, scores, 0.0), axis=-1, keepdims=True)
        masked = jnp.where(lanes == i1, -jnp.inf, biased)
        i2 = jnp.argmax(masked, axis=-1)[:, None]
        w2 = jnp.sum(jnp.where(lanes == i2, scores, 0.0), axis=-1, keepdims=True)
        comb = (jnp.where(lanes == i1, w1, 0.0) +
                jnp.where(lanes == i2, w2, 0.0)) * ROUTE_SCALE
        comb_ref[:, :E] = comb
        comb_ref[:, E:] = jnp.ones((TB, EA - E), jnp.float32)

    xb16 = xb16_ref[...]
    h1 = jax.lax.dot_general(
        xb16, w1_ref[0], (((1,), (1,)), ((), ())),
        preferred_element_type=jnp.float32) + b1_ref[0]
    h3 = jax.lax.dot_general(
        xb16, w3_ref[0], (((1,), (1,)), ((), ())),
        preferred_element_type=jnp.float32) + b3_ref[0]
    h = ((h1 * jax.nn.sigmoid(h1)) * h3).astype(jnp.bfloat16)
    ye = jax.lax.dot_general(
        h, w2_ref[0], (((1,), (1,)), ((), ())),
        preferred_element_type=jnp.float32) + b2_ref[0]
    lane = jax.lax.broadcasted_iota(jnp.int32, (TB, EA), 1)
    col = jnp.sum(jnp.where(lane == e, comb_ref[...], 0.0),
                  axis=1, keepdims=True)
    contrib = ye * col

    @pl.when(e == 0)
    def _init():
        out_ref[...] = contrib

    @pl.when(e != 0)
    def _acc():
        out_ref[...] += contrib


@jax.jit
def _moe(x2, xb16, gate_w, gate_b2, W1a, b1a, W3a, b3a, W2a, b2a):
    T = x2.shape[0]
    grid = (T // TB, EA)
    return pl.pallas_call(
        _moe_body,
        grid=grid,
        in_specs=[
            pl.BlockSpec((TB, D), lambda t, e: (t, 0)),
            pl.BlockSpec((TB, D), lambda t, e: (t, 0)),
            pl.BlockSpec((E, D), lambda t, e: (0, 0)),
            pl.BlockSpec((1, E), lambda t, e: (0, 0)),
            pl.BlockSpec((1, FF, D), lambda t, e: (e, 0, 0)),
            pl.BlockSpec((1, 1, FF), lambda t, e: (e, 0, 0)),
            pl.BlockSpec((1, FF, D), lambda t, e: (e, 0, 0)),
            pl.BlockSpec((1, 1, FF), lambda t, e: (e, 0, 0)),
            pl.BlockSpec((1, D, FF), lambda t, e: (e, 0, 0)),
            pl.BlockSpec((1, 1, D), lambda t, e: (e, 0, 0)),
        ],
        out_specs=pl.BlockSpec((TB, D), lambda t, e: (t, 0)),
        out_shape=jax.ShapeDtypeStruct((T, D), jnp.float32),
        scratch_shapes=[pltpu.VMEM((TB, EA), jnp.float32)],
        compiler_params=pltpu.CompilerParams(
            dimension_semantics=("parallel", "arbitrary"),
        ),
    )(x2, xb16, gate_w, gate_b2, W1a, b1a, W3a, b3a, W2a, b2a)


def kernel(x, gate_w, gate_b, W1, b1, W2, b2, W3, b3,
           Ws1, bs1, Ws2, bs2, Ws3, bs3):
    shape = x.shape
    x2 = x.reshape(-1, D)

    W1a = jnp.concatenate([W1, Ws1.reshape(NS, FF, D)], axis=0)
    b1a = jnp.concatenate([b1, bs1.reshape(NS, FF)], axis=0).reshape(EA, 1, FF)
    W3a = jnp.concatenate([W3, Ws3.reshape(NS, FF, D)], axis=0)
    b3a = jnp.concatenate([b3, bs3.reshape(NS, FF)], axis=0).reshape(EA, 1, FF)
    Ws2s = Ws2.reshape(D, NS, FF).transpose(1, 0, 2)
    W2a = jnp.concatenate([W2, Ws2s], axis=0)
    b2a = jnp.concatenate(
        [b2, bs2[None, :], jnp.zeros((NS - 1, D), jnp.float32)],
        axis=0).reshape(EA, 1, D)

    y = _moe(x2, x2.astype(jnp.bfloat16), gate_w, gate_b.reshape(1, E),
             W1a.astype(jnp.bfloat16), b1a, W3a.astype(jnp.bfloat16), b3a,
             W2a.astype(jnp.bfloat16), b2a)
    return y.reshape(shape)
